# Initial kernel scaffold; baseline (speedup 1.0000x reference)
import jax
import jax.numpy as jnp
from jax.experimental import pallas as pl


def kernel(img_emb, txt_emb, conf_w1, conf_b1, conf_w2, conf_b2, rout_w1, rout_b1, rout_w2, rout_b2, attn_in_w, attn_in_b, attn_out_w, attn_out_b, gate_w, gate_b, ln1_g, ln1_b, ln2_g, ln2_b, ffn_w1, ffn_b1, ffn_w2, ffn_b2, proj_w, proj_b):
    raise NotImplementedError("write your pallas kernel here")


# speedup vs baseline: 1.4824x; 1.4824x over previous
import functools

import jax
import jax.numpy as jnp
from jax.experimental import pallas as pl
from jax.experimental.pallas import tpu as pltpu

_B = 4096
_D = 2048
_H = 16
_DH = _D // _H


def _gelu_f32(x):
    return 0.5 * x * (1.0 + jax.lax.erf(x * (1.0 / jnp.sqrt(jnp.float32(2.0)))))


def _apply_act(y, act):
    if act == "gelu":
        return _gelu_f32(y)
    if act == "sigmoid":
        return jax.nn.sigmoid(y)
    return y


def _mm_body(bnd_ref, x_ref, w_ref, b_ref, *rest, nk, act, f32, out_dtype,
             bm, residual):
    if residual:
        res_ref, o_ref, acc_ref = rest
    else:
        (o_ref, acc_ref) = rest
    m = pl.program_id(0)
    k = pl.program_id(2)
    lo = bnd_ref[0]
    hi = bnd_ref[1]
    valid = jnp.logical_and(m * bm + bm > lo, m * bm < hi)

    @pl.when(valid)
    def _():
        @pl.when(k == 0)
        def _():
            acc_ref[...] = jnp.zeros_like(acc_ref)

        xb = x_ref[...]
        wb = w_ref[...]
        if f32:
            prec = jax.lax.Precision.HIGHEST
        else:
            xb = xb.astype(jnp.bfloat16)
            wb = wb.astype(jnp.bfloat16)
            prec = jax.lax.Precision.DEFAULT
        acc_ref[...] += jax.lax.dot_general(
            xb, wb, (((1,), (1,)), ((), ())),
            preferred_element_type=jnp.float32, precision=prec)

        @pl.when(k == nk - 1)
        def _():
            y = acc_ref[...] + b_ref[...].astype(jnp.float32)
            y = _apply_act(y, act)
            if residual:
                y = y + res_ref[...].astype(jnp.float32)
            o_ref[...] = y.astype(out_dtype)


def _matmul(x, w, b, *, act="none", out_dtype=jnp.bfloat16, f32=False,
            bm=512, bn=1024, bk=2048, bounds=None, residual=None):
    M, K = x.shape
    N, K2 = w.shape
    assert K == K2, (x.shape, w.shape)
    bn = min(bn, N)
    bk = min(bk, K)
    nm, nn, nk = M // bm, N // bn, K // bk
    assert nm * bm == M and nn * bn == N and nk * bk == K

    if bounds is None:
        bounds = jnp.array([0, M], jnp.int32)

    def clamp_m(mi, bnd):
        m_lo = bnd[0] // bm
        m_hi = jnp.maximum(m_lo, (bnd[1] + bm - 1) // bm - 1)
        return jnp.clip(mi, m_lo, m_hi)

    in_specs = [
        pl.BlockSpec((bm, bk), lambda mi, ni, ki, bnd: (clamp_m(mi, bnd), ki)),
        pl.BlockSpec((bn, bk), lambda mi, ni, ki, bnd: (ni, ki)),
        pl.BlockSpec((1, bn), lambda mi, ni, ki, bnd: (0, ni)),
    ]
    args = [x, w, b.reshape(1, N)]
    if residual is not None:
        in_specs.append(
            pl.BlockSpec((bm, bn), lambda mi, ni, ki, bnd: (clamp_m(mi, bnd), ni)))
        args.append(residual)

    grid_spec = pltpu.PrefetchScalarGridSpec(
        num_scalar_prefetch=1,
        grid=(nm, nn, nk),
        in_specs=in_specs,
        out_specs=pl.BlockSpec((bm, bn),
                               lambda mi, ni, ki, bnd: (clamp_m(mi, bnd), ni)),
        scratch_shapes=[pltpu.VMEM((bm, bn), jnp.float32)],
    )
    body = functools.partial(_mm_body, nk=nk, act=act, f32=f32,
                             out_dtype=out_dtype, bm=bm,
                             residual=residual is not None)
    return pl.pallas_call(
        body,
        grid_spec=grid_spec,
        out_shape=jax.ShapeDtypeStruct((M, N), out_dtype),
        compiler_params=pltpu.CompilerParams(
            dimension_semantics=("parallel", "parallel", "arbitrary")),
    )(bounds, *args)


def _decide_body(hc_ref, hr_ref, wc_ref, wr_ref, bc_ref, br_ref, meta_ref, *, bm):
    lc = jnp.dot(hc_ref[...], wc_ref[...],
                 preferred_element_type=jnp.float32) + bc_ref[...]
    lr = jnp.dot(hr_ref[...], wr_ref[...],
                 preferred_element_type=jnp.float32) + br_ref[...]
    l0 = lc[:, 0:1]
    l1 = lc[:, 1:2]
    mx = jnp.maximum(l0, l1)
    e0 = jnp.exp(l0 - mx)
    e1 = jnp.exp(l1 - mx)
    s = e0 + e1
    wv = e0 / s
    wt = e1 / s
    use_hard = jnp.maximum(wv, wt) > 0.6
    r0 = lr[:, 0:1]
    r1 = lr[:, 1:2]
    r2 = lr[:, 2:3]
    grp_hard = jnp.where(jnp.logical_and(r0 >= r1, r0 >= r2), 2.0,
                         jnp.where(r1 >= r2, 3.0, 1.0))
    grp = jnp.where(use_hard, grp_hard, 0.0)
    col = jax.lax.broadcasted_iota(jnp.int32, (bm, 128), 1)
    meta = jnp.where(col == 0, wv,
                     jnp.where(col == 1, wt,
                               jnp.where(col == 2, grp, 0.0)))
    meta_ref[...] = meta


def _decide(hc, hr, wc_pad, wr_pad, bc_pad, br_pad, bm=512):
    nm = _B // bm
    return pl.pallas_call(
        functools.partial(_decide_body, bm=bm),
        grid=(nm,),
        in_specs=[
            pl.BlockSpec((bm, _D), lambda mi: (mi, 0)),
            pl.BlockSpec((bm, _D), lambda mi: (mi, 0)),
            pl.BlockSpec((_D, 128), lambda mi: (0, 0)),
            pl.BlockSpec((_D, 128), lambda mi: (0, 0)),
            pl.BlockSpec((1, 128), lambda mi: (0, 0)),
            pl.BlockSpec((1, 128), lambda mi: (0, 0)),
        ],
        compiler_params=pltpu.CompilerParams(
            dimension_semantics=("parallel",)),
        out_specs=pl.BlockSpec((bm, 128), lambda mi: (mi, 0)),
        out_shape=jax.ShapeDtypeStruct((_B, 128), jnp.float32),
    )(hc, hr, wc_pad, wr_pad, bc_pad, br_pad)


def _attn_body(qkv_i_ref, qkv_t_ref, o_ref, *, bm):
    qi = qkv_i_ref[:, 0:_D].astype(jnp.float32)
    ki = qkv_i_ref[:, _D:2 * _D].astype(jnp.float32)
    vi = qkv_i_ref[:, 2 * _D:3 * _D].astype(jnp.float32)
    qt = qkv_t_ref[:, 0:_D].astype(jnp.float32)
    kt = qkv_t_ref[:, _D:2 * _D].astype(jnp.float32)
    vt = qkv_t_ref[:, 2 * _D:3 * _D].astype(jnp.float32)
    dk = ki - kt
    scale = 1.0 / jnp.sqrt(jnp.float32(_DH))
    d1 = (qi * dk).reshape(bm, _H, _DH).sum(axis=-1) * scale
    d2 = (qt * dk).reshape(bm, _H, _DH).sum(axis=-1) * scale
    a = 0.5 * (jax.nn.sigmoid(d1) + jax.nn.sigmoid(d2))
    dv = (vi - vt).reshape(bm, _H, _DH)
    m = vt.reshape(bm, _H, _DH) + a[:, :, None] * dv
    o_ref[...] = m.reshape(bm, _D).astype(o_ref.dtype)


def _attn_combine(qkv_i, qkv_t, bm=256):
    nm = _B // bm
    return pl.pallas_call(
        functools.partial(_attn_body, bm=bm),
        grid=(nm,),
        in_specs=[
            pl.BlockSpec((bm, 3 * _D), lambda mi: (mi, 0)),
            pl.BlockSpec((bm, 3 * _D), lambda mi: (mi, 0)),
        ],
        out_specs=pl.BlockSpec((bm, _D), lambda mi: (mi, 0)),
        out_shape=jax.ShapeDtypeStruct((_B, _D), jnp.bfloat16),
    )(qkv_i, qkv_t)


def _ln(x, gamma, beta):
    mu = x.mean(axis=-1, keepdims=True)
    d = x - mu
    var = (d * d).mean(axis=-1, keepdims=True)
    return d / jnp.sqrt(var + 1e-5) * gamma + beta


def _mix_body(img_ref, txt_ref, meta_ref, g_ref, attn_ref,
              ln1g_ref, ln1b_ref, ln2g_ref, ln2b_ref,
              soft_ref, both_ref):
    img = img_ref[...]
    txt = txt_ref[...]
    attn = attn_ref[...].astype(jnp.float32)
    wv = meta_ref[:, 0:1]
    wt = meta_ref[:, 1:2]
    g = g_ref[...].astype(jnp.float32)
    base_s = wv * img + wt * txt + attn
    base_b = g * img + (1.0 - g) * txt + attn
    soft_ref[...] = _ln(base_s, ln1g_ref[...], ln1b_ref[...]).astype(soft_ref.dtype)
    both_ref[...] = _ln(base_b, ln2g_ref[...], ln2b_ref[...]).astype(both_ref.dtype)


def _mix_ln(img, txt, meta, g, attn_mean, ln1g, ln1b, ln2g, ln2b, bm=512):
    nm = _B // bm
    row = lambda mi: (mi, 0)
    vec = lambda mi: (0, 0)
    return pl.pallas_call(
        _mix_body,
        grid=(nm,),
        in_specs=[
            pl.BlockSpec((bm, _D), row),
            pl.BlockSpec((bm, _D), row),
            pl.BlockSpec((bm, 128), row),
            pl.BlockSpec((bm, _D), row),
            pl.BlockSpec((bm, _D), row),
            pl.BlockSpec((1, _D), vec),
            pl.BlockSpec((1, _D), vec),
            pl.BlockSpec((1, _D), vec),
            pl.BlockSpec((1, _D), vec),
        ],
        out_specs=[pl.BlockSpec((bm, _D), row), pl.BlockSpec((bm, _D), row)],
        out_shape=[jax.ShapeDtypeStruct((_B, _D), jnp.bfloat16),
                   jax.ShapeDtypeStruct((_B, _D), jnp.bfloat16)],
    )(img, txt, meta, g, attn_mean,
      ln1g.reshape(1, _D), ln1b.reshape(1, _D),
      ln2g.reshape(1, _D), ln2b.reshape(1, _D))


def _select_body(meta_ref, soft_ref, both_ref, pv_ref, pt_ref, o_ref):
    grp = meta_ref[:, 2:3]
    soft = soft_ref[...]
    both = both_ref[...]
    pv = pv_ref[...]
    pt = pt_ref[...]
    hard = jnp.where(grp == 2.0, pv, jnp.where(grp == 3.0, pt, both))
    o_ref[...] = jnp.where(grp == 0.0, soft, hard)


def _select(meta, soft, both, pv, pt, bm=512):
    nm = _B // bm
    row = lambda mi: (mi, 0)
    return pl.pallas_call(
        _select_body,
        grid=(nm,),
        in_specs=[
            pl.BlockSpec((bm, 128), row),
            pl.BlockSpec((bm, _D), row),
            pl.BlockSpec((bm, _D), row),
            pl.BlockSpec((bm, _D), row),
            pl.BlockSpec((bm, _D), row),
        ],
        out_specs=pl.BlockSpec((bm, _D), row),
        out_shape=jax.ShapeDtypeStruct((_B, _D), jnp.float32),
    )(meta, soft, both, pv, pt)


def kernel(img_emb, txt_emb, conf_w1, conf_b1, conf_w2, conf_b2,
           rout_w1, rout_b1, rout_w2, rout_b2,
           attn_in_w, attn_in_b, attn_out_w, attn_out_b,
           gate_w, gate_b, ln1_g, ln1_b, ln2_g, ln2_b,
           ffn_w1, ffn_b1, ffn_w2, ffn_b2, proj_w, proj_b):
    f32 = jnp.float32
    bf16 = jnp.bfloat16
    cat = jnp.concatenate([img_emb, txt_emb], axis=-1)
    img_bf = img_emb.astype(bf16)
    txt_bf = txt_emb.astype(bf16)
    cat_bf = cat.astype(bf16)

    hc = _matmul(cat_bf, conf_w1, conf_b1, act="gelu", out_dtype=bf16,
                 bm=512, bn=1024, bk=2048)
    hr = _matmul(cat_bf, rout_w1, rout_b1, act="gelu", out_dtype=bf16,
                 bm=512, bn=1024, bk=2048)
    wc_pad = jnp.zeros((_D, 128), f32).at[:, 0:2].set(conf_w2.T).astype(bf16)
    wr_pad = jnp.zeros((_D, 128), f32).at[:, 0:3].set(rout_w2.T).astype(bf16)
    bc_pad = jnp.zeros((1, 128), f32).at[0, 0:2].set(conf_b2)
    br_pad = jnp.zeros((1, 128), f32).at[0, 0:3].set(rout_b2)
    meta = _decide(hc, hr, wc_pad, wr_pad, bc_pad, br_pad)

    qkv_i = _matmul(img_bf, attn_in_w, attn_in_b, out_dtype=bf16,
                    bm=512, bn=1024, bk=2048)
    qkv_t = _matmul(txt_bf, attn_in_w, attn_in_b, out_dtype=bf16,
                    bm=512, bn=1024, bk=2048)
    m = _attn_combine(qkv_i, qkv_t)
    attn_mean = _matmul(m, attn_out_w, attn_out_b, out_dtype=bf16,
                        bm=512, bn=1024, bk=2048)

    g = _matmul(cat_bf, gate_w, gate_b, act="sigmoid", out_dtype=bf16,
                bm=512, bn=1024, bk=2048)

    soft_in, both_in = _mix_ln(img_emb, txt_emb, meta, g, attn_mean,
                               ln1_g, ln1_b, ln2_g, ln2_b)

    hs = _matmul(soft_in, ffn_w1, ffn_b1, act="gelu", out_dtype=bf16,
                 bm=512, bn=1024, bk=2048)
    soft_out = _matmul(hs, ffn_w2, ffn_b2, out_dtype=f32, residual=soft_in,
                       bm=512, bn=1024, bk=2048)
    hb = _matmul(both_in, ffn_w1, ffn_b1, act="gelu", out_dtype=bf16,
                 bm=512, bn=1024, bk=2048)
    both_out = _matmul(hb, ffn_w2, ffn_b2, out_dtype=f32, residual=both_in,
                       bm=512, bn=1024, bk=2048)

    pv = _matmul(img_bf, proj_w, proj_b, act="gelu", out_dtype=f32,
                 bm=512, bn=1024, bk=2048)
    pt = _matmul(txt_bf, proj_w, proj_b, act="gelu", out_dtype=f32,
                 bm=512, bn=1024, bk=2048)

    return _select(meta, soft_out, both_out, pv, pt)


# speedup vs baseline: 1.9767x; 1.3334x over previous
import functools

import jax
import jax.numpy as jnp
from jax.experimental import pallas as pl
from jax.experimental.pallas import tpu as pltpu

_B = 4096
_D = 2048
_H = 16
_DH = _D // _H


def _gelu_f32(x):
    return 0.5 * x * (1.0 + jax.lax.erf(x * (1.0 / jnp.sqrt(jnp.float32(2.0)))))


def _apply_act(y, act):
    if act == "gelu":
        return _gelu_f32(y)
    if act == "sigmoid":
        return jax.nn.sigmoid(y)
    return y


def _clamp_m(mi, bnd, bm):
    m_lo = bnd[0] // bm
    m_hi = jnp.maximum(m_lo, (bnd[1] + bm - 1) // bm - 1)
    return jnp.clip(mi, m_lo, m_hi)


def _out_m(mi, bnd, bm, nm):
    m_lo = bnd[0] // bm
    m_hi = jnp.maximum(m_lo, (bnd[1] + bm - 1) // bm - 1)
    return jnp.where(mi < m_lo, 0, jnp.where(mi > m_hi, nm - 1, mi))


def _mm_body(bnd_ref, x_ref, w_ref, b_ref, *rest, nk, act, out_dtype,
             bm, residual):
    if residual:
        res_ref, o_ref, acc_ref = rest
    else:
        (o_ref, acc_ref) = rest
    m = pl.program_id(0)
    k = pl.program_id(2)
    valid = jnp.logical_and(m * bm + bm > bnd_ref[0], m * bm < bnd_ref[1])

    @pl.when(valid)
    def _():
        @pl.when(k == 0)
        def _():
            acc_ref[...] = jnp.zeros_like(acc_ref)

        xb = x_ref[...].astype(jnp.bfloat16)
        wb = w_ref[...].astype(jnp.bfloat16)
        acc_ref[...] += jax.lax.dot_general(
            xb, wb, (((1,), (1,)), ((), ())),
            preferred_element_type=jnp.float32)

        @pl.when(k == nk - 1)
        def _():
            y = acc_ref[...] + b_ref[...].astype(jnp.float32)
            y = _apply_act(y, act)
            if residual:
                y = y + res_ref[...].astype(jnp.float32)
            o_ref[...] = y.astype(out_dtype)


def _matmul(x, w, b, *, act="none", out_dtype=jnp.bfloat16,
            bm=512, bn=1024, bk=2048, bounds=None, residual=None, x_off=0):
    M = x.shape[0]
    N, K = w.shape
    bn = min(bn, N)
    bk = min(bk, K)
    nm, nn, nk = M // bm, N // bn, K // bk
    assert nm * bm == M and nn * bn == N and nk * bk == K and x_off % bk == 0
    koff = x_off // bk

    if bounds is None:
        bounds = jnp.array([0, M], jnp.int32)

    in_specs = [
        pl.BlockSpec((bm, bk),
                     lambda mi, ni, ki, bnd: (_clamp_m(mi, bnd, bm), ki + koff)),
        pl.BlockSpec((bn, bk), lambda mi, ni, ki, bnd: (ni, ki)),
        pl.BlockSpec((1, bn), lambda mi, ni, ki, bnd: (0, ni)),
    ]
    args = [x, w, b.reshape(1, N)]
    if residual is not None:
        in_specs.append(
            pl.BlockSpec((bm, bn),
                         lambda mi, ni, ki, bnd: (_clamp_m(mi, bnd, bm), ni)))
        args.append(residual)

    grid_spec = pltpu.PrefetchScalarGridSpec(
        num_scalar_prefetch=1,
        grid=(nm, nn, nk),
        in_specs=in_specs,
        out_specs=pl.BlockSpec(
            (bm, bn), lambda mi, ni, ki, bnd: (_out_m(mi, bnd, bm, nm), ni)),
        scratch_shapes=[pltpu.VMEM((bm, bn), jnp.float32)],
    )
    body = functools.partial(_mm_body, nk=nk, act=act,
                             out_dtype=out_dtype, bm=bm,
                             residual=residual is not None)
    return pl.pallas_call(
        body,
        grid_spec=grid_spec,
        out_shape=jax.ShapeDtypeStruct((M, N), out_dtype),
        compiler_params=pltpu.CompilerParams(
            dimension_semantics=("parallel", "parallel", "arbitrary")),
    )(bounds, *args)


def _decide_body(hc_ref, hr_ref, wc_ref, wr_ref, bc_ref, br_ref, meta_ref, *, bm):
    lc = jnp.dot(hc_ref[...], wc_ref[...],
                 preferred_element_type=jnp.float32) + bc_ref[...]
    lr = jnp.dot(hr_ref[...], wr_ref[...],
                 preferred_element_type=jnp.float32) + br_ref[...]
    l0 = lc[:, 0:1]
    l1 = lc[:, 1:2]
    mx = jnp.maximum(l0, l1)
    e0 = jnp.exp(l0 - mx)
    e1 = jnp.exp(l1 - mx)
    s = e0 + e1
    wv = e0 / s
    wt = e1 / s
    use_hard = jnp.maximum(wv, wt) > 0.6
    r0 = lr[:, 0:1]
    r1 = lr[:, 1:2]
    r2 = lr[:, 2:3]
    grp_hard = jnp.where(jnp.logical_and(r0 >= r1, r0 >= r2), 2.0,
                         jnp.where(r1 >= r2, 3.0, 1.0))
    grp = jnp.where(use_hard, grp_hard, 0.0)
    col = jax.lax.broadcasted_iota(jnp.int32, (bm, 128), 1)
    meta = jnp.where(col == 0, wv,
                     jnp.where(col == 1, wt,
                               jnp.where(col == 2, grp, 0.0)))
    meta_ref[...] = meta


def _decide(hc, hr, wc_pad, wr_pad, bc_pad, br_pad, bm=512):
    nm = _B // bm
    return pl.pallas_call(
        functools.partial(_decide_body, bm=bm),
        grid=(nm,),
        in_specs=[
            pl.BlockSpec((bm, _D), lambda mi: (mi, 0)),
            pl.BlockSpec((bm, _D), lambda mi: (mi, 0)),
            pl.BlockSpec((_D, 128), lambda mi: (0, 0)),
            pl.BlockSpec((_D, 128), lambda mi: (0, 0)),
            pl.BlockSpec((1, 128), lambda mi: (0, 0)),
            pl.BlockSpec((1, 128), lambda mi: (0, 0)),
        ],
        out_specs=pl.BlockSpec((bm, 128), lambda mi: (mi, 0)),
        out_shape=jax.ShapeDtypeStruct((_B, 128), jnp.float32),
        compiler_params=pltpu.CompilerParams(
            dimension_semantics=("parallel",)),
    )(hc, hr, wc_pad, wr_pad, bc_pad, br_pad)


def _attn_body(bnd_ref, qkv_i_ref, qkv_t_ref, o_ref, *, bm):
    m = pl.program_id(0)
    valid = jnp.logical_and(m * bm + bm > bnd_ref[0], m * bm < bnd_ref[1])

    @pl.when(valid)
    def _():
        qi = qkv_i_ref[:, 0:_D].astype(jnp.float32)
        ki = qkv_i_ref[:, _D:2 * _D].astype(jnp.float32)
        vi = qkv_i_ref[:, 2 * _D:3 * _D].astype(jnp.float32)
        qt = qkv_t_ref[:, 0:_D].astype(jnp.float32)
        kt = qkv_t_ref[:, _D:2 * _D].astype(jnp.float32)
        vt = qkv_t_ref[:, 2 * _D:3 * _D].astype(jnp.float32)
        dk = ki - kt
        scale = 1.0 / jnp.sqrt(jnp.float32(_DH))
        d1 = (qi * dk).reshape(bm, _H, _DH).sum(axis=-1) * scale
        d2 = (qt * dk).reshape(bm, _H, _DH).sum(axis=-1) * scale
        a = 0.5 * (jax.nn.sigmoid(d1) + jax.nn.sigmoid(d2))
        dv = (vi - vt).reshape(bm, _H, _DH)
        mm = vt.reshape(bm, _H, _DH) + a[:, :, None] * dv
        o_ref[...] = mm.reshape(bm, _D).astype(o_ref.dtype)


def _attn_combine(qkv_i, qkv_t, bounds, bm=256):
    nm = _B // bm
    grid_spec = pltpu.PrefetchScalarGridSpec(
        num_scalar_prefetch=1,
        grid=(nm,),
        in_specs=[
            pl.BlockSpec((bm, 3 * _D), lambda mi, bnd: (_clamp_m(mi, bnd, bm), 0)),
            pl.BlockSpec((bm, 3 * _D), lambda mi, bnd: (_clamp_m(mi, bnd, bm), 0)),
        ],
        out_specs=pl.BlockSpec((bm, _D), lambda mi, bnd: (_out_m(mi, bnd, bm, nm), 0)),
    )
    return pl.pallas_call(
        functools.partial(_attn_body, bm=bm),
        grid_spec=grid_spec,
        out_shape=jax.ShapeDtypeStruct((_B, _D), jnp.bfloat16),
        compiler_params=pltpu.CompilerParams(
            dimension_semantics=("parallel",)),
    )(bounds, qkv_i, qkv_t)


def _mixs_body(bnd_ref, img_ref, txt_ref, meta_ref, g_ref, attn_ref,
               ln1g_ref, ln1b_ref, ln2g_ref, ln2b_ref, o_ref, *, bm):
    m = pl.program_id(0)
    n0 = bnd_ref[2]
    valid = jnp.logical_and(m * bm + bm > bnd_ref[0], m * bm < bnd_ref[1])

    @pl.when(valid)
    def _():
        rows = m * bm + jax.lax.broadcasted_iota(jnp.int32, (bm, 1), 0)
        soft = rows < n0
        img = img_ref[...]
        txt = txt_ref[...]
        attn = attn_ref[...].astype(jnp.float32)
        wv = meta_ref[:, 0:1]
        wt = meta_ref[:, 1:2]
        g = g_ref[...].astype(jnp.float32)
        base = jnp.where(soft, wv * img + wt * txt,
                         g * img + (1.0 - g) * txt) + attn
        gamma = jnp.where(soft, ln1g_ref[...], ln2g_ref[...])
        beta = jnp.where(soft, ln1b_ref[...], ln2b_ref[...])
        mu = base.mean(axis=-1, keepdims=True)
        dx = base - mu
        var = (dx * dx).mean(axis=-1, keepdims=True)
        o_ref[...] = (dx / jnp.sqrt(var + 1e-5) * gamma + beta).astype(o_ref.dtype)


def _mix_ln_sorted(cat_g, meta_g, g, attn_mean, ln1g, ln1b, ln2g, ln2b,
                   bounds, bm=256):
    nm = _B // bm
    def rowm(mi, bnd):
        return (_clamp_m(mi, bnd, bm), 0)
    def rowm_txt(mi, bnd):
        return (_clamp_m(mi, bnd, bm), 1)
    vec = lambda mi, bnd: (0, 0)
    grid_spec = pltpu.PrefetchScalarGridSpec(
        num_scalar_prefetch=1,
        grid=(nm,),
        in_specs=[
            pl.BlockSpec((bm, _D), rowm),
            pl.BlockSpec((bm, _D), rowm_txt),
            pl.BlockSpec((bm, 128), rowm),
            pl.BlockSpec((bm, _D), rowm),
            pl.BlockSpec((bm, _D), rowm),
            pl.BlockSpec((1, _D), vec),
            pl.BlockSpec((1, _D), vec),
            pl.BlockSpec((1, _D), vec),
            pl.BlockSpec((1, _D), vec),
        ],
        out_specs=pl.BlockSpec((bm, _D),
                               lambda mi, bnd: (_out_m(mi, bnd, bm, nm), 0)),
    )
    return pl.pallas_call(
        functools.partial(_mixs_body, bm=bm),
        grid_spec=grid_spec,
        out_shape=jax.ShapeDtypeStruct((_B, _D), jnp.bfloat16),
        compiler_params=pltpu.CompilerParams(
            dimension_semantics=("parallel",)),
    )(bounds, cat_g, cat_g, meta_g, g, attn_mean,
      ln1g.reshape(1, _D), ln1b.reshape(1, _D),
      ln2g.reshape(1, _D), ln2b.reshape(1, _D))


def _fin_body(bnd_ref, img_ref, txt_ref, ffn_ref, w_ref, b_ref, o_ref, *, bm):
    m = pl.program_id(0)
    n01 = bnd_ref[0]
    n012 = bnd_ref[1]
    rows = m * bm + jax.lax.broadcasted_iota(jnp.int32, (bm, 1), 0)
    need_proj = m * bm + bm > n01

    @pl.when(need_proj)
    def _():
        sel = jnp.where(rows < n012, img_ref[...], txt_ref[...])
        y = jax.lax.dot_general(
            sel.astype(jnp.bfloat16), w_ref[...].astype(jnp.bfloat16),
            (((1,), (1,)), ((), ())), preferred_element_type=jnp.float32)
        y = _gelu_f32(y + b_ref[...])
        o_ref[...] = jnp.where(rows < n01, ffn_ref[...], y)

    @pl.when(jnp.logical_not(need_proj))
    def _():
        o_ref[...] = ffn_ref[...]


def _finish(cat_g, ffn_out, proj_w, proj_b, bounds, bm=256, bn=1024):
    nm = _B // bm
    nn = _D // bn
    grid_spec = pltpu.PrefetchScalarGridSpec(
        num_scalar_prefetch=1,
        grid=(nm, nn),
        in_specs=[
            pl.BlockSpec((bm, _D), lambda mi, ni, bnd: (mi, 0)),
            pl.BlockSpec((bm, _D), lambda mi, ni, bnd: (mi, 1)),
            pl.BlockSpec((bm, bn), lambda mi, ni, bnd: (mi, ni)),
            pl.BlockSpec((bn, _D), lambda mi, ni, bnd: (ni, 0)),
            pl.BlockSpec((1, bn), lambda mi, ni, bnd: (0, ni)),
        ],
        out_specs=pl.BlockSpec((bm, bn), lambda mi, ni, bnd: (mi, ni)),
    )
    return pl.pallas_call(
        functools.partial(_fin_body, bm=bm),
        grid_spec=grid_spec,
        out_shape=jax.ShapeDtypeStruct((_B, _D), jnp.float32),
        compiler_params=pltpu.CompilerParams(
            dimension_semantics=("parallel", "parallel")),
    )(bounds, cat_g, cat_g, ffn_out, proj_w, proj_b.reshape(1, _D))


def kernel(img_emb, txt_emb, conf_w1, conf_b1, conf_w2, conf_b2,
           rout_w1, rout_b1, rout_w2, rout_b2,
           attn_in_w, attn_in_b, attn_out_w, attn_out_b,
           gate_w, gate_b, ln1_g, ln1_b, ln2_g, ln2_b,
           ffn_w1, ffn_b1, ffn_w2, ffn_b2, proj_w, proj_b):
    f32 = jnp.float32
    bf16 = jnp.bfloat16
    i32 = jnp.int32
    cat = jnp.concatenate([img_emb, txt_emb], axis=-1)
    cat_bf = cat.astype(bf16)

    hc = _matmul(cat_bf, conf_w1, conf_b1, act="gelu", out_dtype=bf16)
    hr = _matmul(cat_bf, rout_w1, rout_b1, act="gelu", out_dtype=bf16)
    wc_pad = jnp.zeros((_D, 128), f32).at[:, 0:2].set(conf_w2.T).astype(bf16)
    wr_pad = jnp.zeros((_D, 128), f32).at[:, 0:3].set(rout_w2.T).astype(bf16)
    bc_pad = jnp.zeros((1, 128), f32).at[0, 0:2].set(conf_b2)
    br_pad = jnp.zeros((1, 128), f32).at[0, 0:3].set(rout_b2)
    meta = _decide(hc, hr, wc_pad, wr_pad, bc_pad, br_pad)

    group = meta[:, 2].astype(i32)
    order = jnp.argsort(group)
    n0 = jnp.sum(group == 0).astype(i32)
    n1 = jnp.sum(group == 1).astype(i32)
    n2 = jnp.sum(group == 2).astype(i32)
    n01 = n0 + n1
    n012 = n01 + n2
    cat_g = cat[order]
    meta_g = meta[order]

    z = jnp.int32(0)
    b_attn = jnp.stack([z, n01])
    b_gate = jnp.stack([n0, n01])
    b_mix = jnp.stack([z, n01, n0])
    b_fin = jnp.stack([n01, n012])

    qkv_i = _matmul(cat_g, attn_in_w, attn_in_b, bounds=b_attn, x_off=0)
    qkv_t = _matmul(cat_g, attn_in_w, attn_in_b, bounds=b_attn, x_off=_D)
    m = _attn_combine(qkv_i, qkv_t, b_attn)
    attn_mean = _matmul(m, attn_out_w, attn_out_b, bounds=b_attn)

    g = _matmul(cat_g, gate_w, gate_b, act="sigmoid", bounds=b_gate)

    mix = _mix_ln_sorted(cat_g, meta_g, g, attn_mean,
                         ln1_g, ln1_b, ln2_g, ln2_b, b_mix)
    hs = _matmul(mix, ffn_w1, ffn_b1, act="gelu", bounds=b_attn)
    ffn_out = _matmul(hs, ffn_w2, ffn_b2, out_dtype=f32, residual=mix,
                      bounds=b_attn)

    out_sorted = _finish(cat_g, ffn_out, proj_w, proj_b, b_fin)
    inv = jnp.argsort(order)
    return out_sorted[inv]
